# Initial kernel scaffold; baseline (speedup 1.0000x reference)
import jax
import jax.numpy as jnp
from jax.experimental import pallas as pl


def kernel(token_ids, embedding):
    raise NotImplementedError("write your pallas kernel here")


# speedup vs baseline: 1.8845x; 1.8845x over previous
import functools

import jax
import jax.numpy as jnp
from jax import lax
from jax.experimental import pallas as pl
from jax.experimental.pallas import tpu as pltpu
from jax.experimental.pallas import tpu_sc as plsc

_IDX_PER_DMA = 128
_C = 640


@functools.lru_cache(maxsize=None)
def _make_gather(B: int, D: int):
    info = plsc.get_sparse_core_info()
    NC, NS = info.num_cores, info.num_subcores
    NW = NC * NS
    assert B % NW == 0
    b_per_w = B // NW
    C = _C
    K = C // _IDX_PER_DMA
    G = b_per_w // C
    R = b_per_w // _IDX_PER_DMA
    assert b_per_w % C == 0 and C % _IDX_PER_DMA == 0 and G % 2 == 0
    assert (R % 8 == 0) and (B // _IDX_PER_DMA) % 8 == 0
    mesh = plsc.VectorSubcoreMesh(core_axis_name="c", subcore_axis_name="s")

    @functools.partial(
        pl.kernel,
        mesh=mesh,
        compiler_params=pltpu.CompilerParams(use_tc_tiling_on_sc=False),
        out_type=jax.ShapeDtypeStruct((B, D), jnp.float32),
        scratch_types=[
            pltpu.VMEM((R, _IDX_PER_DMA), jnp.int32),
            pltpu.VMEM((C, D), jnp.float32),
            pltpu.VMEM((C, D), jnp.float32),
            pltpu.SemaphoreType.DMA,
            pltpu.SemaphoreType.DMA,
            pltpu.SemaphoreType.DMA,
            pltpu.SemaphoreType.DMA,
        ],
    )
    def gather_kernel(table_hbm, idx_hbm, out_hbm,
                      idx_v, rows0, rows1, gsem0, gsem1, wsem0, wsem1):
        wid = lax.axis_index("s") * NC + lax.axis_index("c")
        base = wid * b_per_w
        rows = (rows0, rows1)
        gsem = (gsem0, gsem1)
        wsem = (wsem0, wsem1)

        pltpu.sync_copy(idx_hbm.at[pl.ds(wid * R, R)], idx_v)

        def start_gather(gi, b):
            for j in range(K):
                pltpu.async_copy(
                    table_hbm.at[idx_v.at[gi * K + j]],
                    rows[b].at[pl.ds(j * _IDX_PER_DMA, _IDX_PER_DMA)],
                    gsem[b],
                )

        def wait_gather(b):
            pltpu.make_async_copy(table_hbm.at[pl.ds(0, C)], rows[b], gsem[b]).wait()

        def start_write(gi, b):
            pltpu.async_copy(rows[b], out_hbm.at[pl.ds(base + gi * C, C)], wsem[b])

        def wait_write(b):
            pltpu.make_async_copy(rows[b], out_hbm.at[pl.ds(0, C)], wsem[b]).wait()

        start_gather(0, 0)

        @pl.loop(0, G, step=2)
        def _pair(g):
            for b in (0, 1):
                gi = g + b
                b1 = 1 - b

                @pl.when(gi >= 1)
                def _():
                    wait_write(b1)

                @pl.when(gi + 1 < G)
                def _():
                    start_gather(gi + 1, b1)

                wait_gather(b)
                start_write(gi, b)

        wait_write(1)

    return gather_kernel


def kernel(token_ids, embedding):
    B = token_ids.size
    D = embedding.shape[1]
    idx2 = token_ids.reshape(B // _IDX_PER_DMA, _IDX_PER_DMA).astype(jnp.int32)
    out = _make_gather(B, D)(embedding, idx2)
    return out.reshape(*token_ids.shape, D)
